# Initial kernel scaffold; baseline (speedup 1.0000x reference)
import functools
import jax
import jax.numpy as jnp
from jax import lax
from jax.experimental import pallas as pl
from jax.experimental.pallas import tpu as pltpu
from jax.experimental.pallas import tpu_sc as plsc

NUM_TOKENS = 16384 * 50
DIM = 64
NC, NS = 2, 16
NW = NC * NS
B_PER_W = NUM_TOKENS // NW
CHUNK = 128
SPT = B_PER_W // CHUNK
NBUF = 8
OUTER = SPT // NBUF


@functools.partial(
    pl.kernel,
    mesh=plsc.VectorSubcoreMesh(core_axis_name="c", subcore_axis_name="s"),
    out_type=jax.ShapeDtypeStruct((NUM_TOKENS, DIM), jnp.float32),
    scratch_types=(
        [pltpu.VMEM((SPT, CHUNK), jnp.int32)]
        + [pltpu.VMEM((CHUNK, DIM), jnp.float32) for _ in range(NBUF)]
        + [pltpu.SemaphoreType.DMA for _ in range(2 * NBUF)]
    ),
)
def _embedding_gather(table_hbm, idx_hbm, out_hbm, idx_v, *bufs_and_sems):
    rows = bufs_and_sems[:NBUF]
    gsem = bufs_and_sems[NBUF:2 * NBUF]
    osem = bufs_and_sems[2 * NBUF:]

    wid = lax.axis_index("s") * NC + lax.axis_index("c")
    base = wid * B_PER_W

    pltpu.sync_copy(idx_hbm.at[wid], idx_v)

    def gather_start(j, b):
        pltpu.async_copy(table_hbm.at[idx_v.at[j]], rows[b], gsem[b])

    def gather_wait(b):
        pltpu.make_async_copy(table_hbm.at[idx_v.at[0]], rows[b], gsem[b]).wait()

    def out_start(j, b):
        pltpu.async_copy(rows[b], out_hbm.at[pl.ds(base + j * CHUNK, CHUNK)],
                         osem[b])

    def out_wait(b):
        pltpu.make_async_copy(rows[b], out_hbm.at[pl.ds(base, CHUNK)],
                              osem[b]).wait()

    for b in range(NBUF):
        gather_start(b, b)

    def body(i, carry):
        for b in range(NBUF):
            j = i * NBUF + b
            gather_wait(b)
            out_start(j, b)
            nxt = j + NBUF

            @pl.when(nxt < SPT)
            def _():
                out_wait(b)
                gather_start(nxt, b)

        return carry

    lax.fori_loop(0, OUTER, body, 0)

    for b in range(NBUF):
        out_wait(b)


def kernel(token_ids, weight):
    idx = token_ids.reshape(NW, SPT, CHUNK).astype(jnp.int32)
    out = _embedding_gather(weight, idx)
    return out.reshape(token_ids.shape + (DIM,))


# speedup vs baseline: 1.8896x; 1.8896x over previous
import functools
import jax
import jax.numpy as jnp
from jax import lax
from jax.experimental import pallas as pl
from jax.experimental.pallas import tpu as pltpu
from jax.experimental.pallas import tpu_sc as plsc

NUM_TOKENS = 16384 * 50
DIM = 64
NC, NS = 2, 16
NW = NC * NS
B_PER_W = NUM_TOKENS // NW
CHUNK = 128
SPT = B_PER_W // CHUNK
NBUF = 8
OUTER = SPT // NBUF


@functools.partial(
    pl.kernel,
    mesh=plsc.VectorSubcoreMesh(core_axis_name="c", subcore_axis_name="s"),
    out_type=jax.ShapeDtypeStruct((NUM_TOKENS, DIM), jnp.float32),
    scratch_types=(
        [pltpu.VMEM((SPT, CHUNK), jnp.int32)]
        + [pltpu.VMEM((CHUNK, DIM), jnp.float32) for _ in range(NBUF)]
        + [pltpu.SemaphoreType.DMA for _ in range(2 * NBUF)]
    ),
    compiler_params=pltpu.CompilerParams(use_tc_tiling_on_sc=False),
)
def _embedding_gather(table_hbm, idx_hbm, out_hbm, idx_v, *bufs_and_sems):
    rows = bufs_and_sems[:NBUF]
    gsem = bufs_and_sems[NBUF:2 * NBUF]
    osem = bufs_and_sems[2 * NBUF:]

    wid = lax.axis_index("s") * NC + lax.axis_index("c")
    base = wid * B_PER_W

    pltpu.sync_copy(idx_hbm.at[wid], idx_v)

    def gather_start(j, b):
        pltpu.async_copy(table_hbm.at[idx_v.at[j]], rows[b], gsem[b])

    def gather_wait(b):
        pltpu.make_async_copy(table_hbm.at[idx_v.at[0]], rows[b], gsem[b]).wait()

    def out_start(j, b):
        pltpu.async_copy(rows[b], out_hbm.at[pl.ds(base + j * CHUNK, CHUNK)],
                         osem[b])

    def out_wait(b):
        pltpu.make_async_copy(rows[b], out_hbm.at[pl.ds(base, CHUNK)],
                              osem[b]).wait()

    for b in range(NBUF):
        gather_start(b, b)

    def body(i, carry):
        for b in range(NBUF):
            j = i * NBUF + b
            gather_wait(b)
            out_start(j, b)
            nxt = j + NBUF

            @pl.when(nxt < SPT)
            def _():
                out_wait(b)
                gather_start(nxt, b)

        return carry

    lax.fori_loop(0, OUTER, body, 0)

    for b in range(NBUF):
        out_wait(b)


def kernel(token_ids, weight):
    idx = token_ids.reshape(NW, SPT, CHUNK).astype(jnp.int32)
    out = _embedding_gather(weight, idx)
    return out.reshape(token_ids.shape + (DIM,))
